# baseline (speedup 1.0000x reference)
import functools

import jax
import jax.numpy as jnp
from jax.experimental import pallas as pl
from jax.experimental.pallas import tpu as pltpu
from jax.experimental.pallas import tpu_sc as plsc

TEXT_DIM = 512
AUDIO_DIM = 128
H = 128
OUT = 2 * H
T = 2048
BATCH = 8
WINDOW = 20
HOPS = 3
AIM = 256
NC = 7

BT = 256
NB = T // BT
BTA = 256
NBA = T // BTA
GBT = 256
NGB = T // GBT
PAD = 32


def _xw_kernel(xt_ref, xa_ref, Wt_ref, Wa_ref, b_ref, outf_ref, outb_ref):
    for b in range(BATCH):
        g = jnp.dot(xt_ref[b], Wt_ref[...], preferred_element_type=jnp.float32)
        g = g + jnp.dot(xa_ref[b], Wa_ref[...], preferred_element_type=jnp.float32)
        g = g + b_ref[...]
        outf_ref[:, b, :] = g[:, : 4 * H]
        outb_ref[:, b, :] = g[:, 4 * H:]


def _lstm_kernel(xwf_ref, xwb_ref, Whf_ref, Whb_ref, hf_ref, hb_ref,
                 hf_sc, hb_sc, cf_sc, cb_sc):
    i = pl.program_id(0)

    @pl.when(i == 0)
    def _():
        hf_sc[...] = jnp.zeros_like(hf_sc)
        hb_sc[...] = jnp.zeros_like(hb_sc)
        cf_sc[...] = jnp.zeros_like(cf_sc)
        cb_sc[...] = jnp.zeros_like(cb_sc)

    def sig(x):
        return 0.5 * jnp.tanh(0.5 * x) + 0.5

    def gates(g, c):
        ii = sig(g[:, 0:H])
        ff = sig(g[:, H:2 * H])
        uu = jnp.tanh(g[:, 2 * H:3 * H])
        oo = sig(g[:, 3 * H:4 * H])
        cn = ff * c + ii * uu
        hn = oo * jnp.tanh(cn)
        return hn, cn

    def body(k, carry):
        h_f, h_b, c_f, c_b = carry
        gf = jnp.dot(h_f.astype(jnp.bfloat16), Whf_ref[...],
                     preferred_element_type=jnp.float32) + xwf_ref[k]
        gb = jnp.dot(h_b.astype(jnp.bfloat16), Whb_ref[...],
                     preferred_element_type=jnp.float32) + xwb_ref[BT - 1 - k]
        hf_new, cf_new = gates(gf, c_f)
        hb_new, cb_new = gates(gb, c_b)
        hf_ref[k] = hf_new
        hb_ref[BT - 1 - k] = hb_new
        return (hf_new, hb_new, cf_new, cb_new)

    carry = (hf_sc[...], hb_sc[...], cf_sc[...], cb_sc[...])
    h_f, h_b, c_f, c_b = jax.lax.fori_loop(0, BT, body, carry, unroll=16)
    hf_sc[...] = h_f
    hb_sc[...] = h_b
    cf_sc[...] = c_f
    cb_sc[...] = c_b


GB = 256
HALO = 64
ROWS = GB + HALO
FH = 128
NTB = T // GB
UNITS_PER_WORKER = (BATCH * NTB * 2) // 32


def _grn_sc_kernel(hf_hbm, hb_hbm, out_hbm, bufA, bufB, acc, invd):
    cid = jax.lax.axis_index("c")
    sid = jax.lax.axis_index("s")
    wid = sid * 2 + cid

    for u in range(UNITS_PER_WORKER):
        unit = wid * UNITS_PER_WORKER + u
        half = unit % 2
        tb = (unit // 2) % NTB
        b = unit // (2 * NTB)
        r0 = tb * GB

        @pl.when(jnp.logical_and(half == 0, tb == 0))
        def _():
            pltpu.sync_copy(hf_hbm.at[pl.ds(0, GB), b, :],
                            bufA.at[pl.ds(HALO, GB), :])

        @pl.when(jnp.logical_and(half == 1, tb == 0))
        def _():
            pltpu.sync_copy(hb_hbm.at[pl.ds(0, GB), b, :],
                            bufA.at[pl.ds(HALO, GB), :])

        @pl.when(jnp.logical_and(half == 0, tb != 0))
        def _():
            pltpu.sync_copy(hf_hbm.at[pl.ds(r0 - HALO, ROWS), b, :],
                            bufA.at[:])

        @pl.when(jnp.logical_and(half == 1, tb != 0))
        def _():
            pltpu.sync_copy(hb_hbm.at[pl.ds(r0 - HALO, ROWS), b, :],
                            bufA.at[:])

        @pl.when(tb == 0)
        def _():
            def zrow(j, _):
                for c in range(FH // 16):
                    bufA[j, pl.ds(c * 16, 16)] = jnp.zeros((16,), jnp.float32)
                return 0
            jax.lax.fori_loop(0, HALO, zrow, 0)

        def fill(j, _):
            invd[j] = jnp.float32(1.0 / (WINDOW + 1))
            return 0
        jax.lax.fori_loop(0, ROWS, fill, 0)

        @pl.when(tb == 0)
        def _():
            for g in range(WINDOW):
                invd[HALO + g] = jnp.float32(1.0 / (g + 1))

        def initacc(j, _):
            for c in range(FH // 16):
                acc[j, pl.ds(c * 16, 16)] = bufA[HALO + j, pl.ds(c * 16, 16)]
            return 0
        jax.lax.fori_loop(0, GB, initacc, 0, unroll=4)

        bufs = [bufA, bufB, bufA, bufB]
        for hop in range(HOPS):
            X, Y = bufs[hop], bufs[hop + 1]

            def presum(j, S):
                return tuple(S[c] + X[j, pl.ds(c * 16, 16)]
                             for c in range(FH // 16))

            S0 = tuple(jnp.zeros((16,), jnp.float32) for _ in range(FH // 16))
            S = jax.lax.fori_loop(0, WINDOW, presum, S0)

            def row(j, S):
                inv = invd[j]
                out = []
                for c in range(FH // 16):
                    s = S[c] + X[j, pl.ds(c * 16, 16)]
                    Y[j, pl.ds(c * 16, 16)] = s * inv
                    out.append(s - X[j - WINDOW, pl.ds(c * 16, 16)])
                return tuple(out)
            jax.lax.fori_loop(WINDOW, ROWS, row, S, unroll=4)

            def zy(j, _):
                for c in range(FH // 16):
                    Y[j, pl.ds(c * 16, 16)] = jnp.zeros((16,), jnp.float32)
                return 0
            jax.lax.fori_loop(0, WINDOW, zy, 0)

            def addacc(j, _):
                for c in range(FH // 16):
                    acc[j, pl.ds(c * 16, 16)] += Y[HALO + j, pl.ds(c * 16, 16)]
                return 0
            jax.lax.fori_loop(0, GB, addacc, 0, unroll=4)

        pltpu.sync_copy(acc.at[:],
                        out_hbm.at[pl.ds(r0, GB), b, pl.ds(half * FH, FH)])


_grn_sc = functools.partial(
    pl.kernel,
    mesh=plsc.VectorSubcoreMesh(core_axis_name="c", subcore_axis_name="s"),
    out_type=jax.ShapeDtypeStruct((T, BATCH, OUT), jnp.float32),
    scratch_types=[
        pltpu.VMEM((ROWS, FH), jnp.float32),
        pltpu.VMEM((ROWS, FH), jnp.float32),
        pltpu.VMEM((GB, FH), jnp.float32),
        pltpu.SMEM((ROWS,), jnp.float32),
    ],
)(_grn_sc_kernel)


def _aim_kernel(hf_ref, hb_ref, graph_ref, WgL_ref, WgG_ref, Wx_ref, Wgr_ref,
                Wc_ref, bg_ref, bfu_ref, bc_ref, out_ref):
    lstm = jnp.concatenate([hf_ref[...], hb_ref[...]], axis=1)
    graph = graph_ref[...] * (1.0 / (HOPS + 1))

    gate = jax.nn.sigmoid(
        jnp.dot(lstm, WgL_ref[...], preferred_element_type=jnp.float32)
        + jnp.dot(graph, WgG_ref[...], preferred_element_type=jnp.float32)
        + bg_ref[...])
    fused = jnp.tanh(
        gate * jnp.dot(lstm, Wx_ref[...], preferred_element_type=jnp.float32)
        + (1.0 - gate) * jnp.dot(graph, Wgr_ref[...], preferred_element_type=jnp.float32)
        + bfu_ref[...])
    out_ref[0] = jnp.dot(fused, Wc_ref[...], preferred_element_type=jnp.float32) + bc_ref[...]


def kernel(text_embeds, audio_feats, speaker_ids, W_ih_f, W_hh_f, b_f,
           W_ih_b, W_hh_b, b_b, Wg, bg, Wx, Wgr, bf, Wc, bc):
    f32 = jnp.float32
    Wt = jnp.concatenate([W_ih_f[:, :TEXT_DIM], W_ih_b[:, :TEXT_DIM]], axis=0).T
    Wa = jnp.concatenate([W_ih_f[:, TEXT_DIM:], W_ih_b[:, TEXT_DIM:]], axis=0).T
    bcat = jnp.concatenate([b_f, b_b]).reshape(1, 8 * H)
    Whf = W_hh_f.T.astype(jnp.bfloat16)
    Whb = W_hh_b.T.astype(jnp.bfloat16)

    xwf, xwb = pl.pallas_call(
        _xw_kernel,
        grid=(NBA,),
        in_specs=[
            pl.BlockSpec((BATCH, BTA, TEXT_DIM), lambda i: (0, i, 0)),
            pl.BlockSpec((BATCH, BTA, AUDIO_DIM), lambda i: (0, i, 0)),
            pl.BlockSpec((TEXT_DIM, 8 * H), lambda i: (0, 0)),
            pl.BlockSpec((AUDIO_DIM, 8 * H), lambda i: (0, 0)),
            pl.BlockSpec((1, 8 * H), lambda i: (0, 0)),
        ],
        out_specs=[
            pl.BlockSpec((BTA, BATCH, 4 * H), lambda i: (i, 0, 0)),
            pl.BlockSpec((BTA, BATCH, 4 * H), lambda i: (i, 0, 0)),
        ],
        out_shape=[
            jax.ShapeDtypeStruct((T, BATCH, 4 * H), f32),
            jax.ShapeDtypeStruct((T, BATCH, 4 * H), f32),
        ],
    )(text_embeds, audio_feats, Wt, Wa, bcat)

    hf, hb = pl.pallas_call(
        _lstm_kernel,
        grid=(NB,),
        in_specs=[
            pl.BlockSpec((BT, BATCH, 4 * H), lambda i: (i, 0, 0)),
            pl.BlockSpec((BT, BATCH, 4 * H), lambda i: (NB - 1 - i, 0, 0)),
            pl.BlockSpec((H, 4 * H), lambda i: (0, 0)),
            pl.BlockSpec((H, 4 * H), lambda i: (0, 0)),
        ],
        out_specs=[
            pl.BlockSpec((BT, BATCH, H), lambda i: (i, 0, 0)),
            pl.BlockSpec((BT, BATCH, H), lambda i: (NB - 1 - i, 0, 0)),
        ],
        out_shape=[
            jax.ShapeDtypeStruct((T, BATCH, H), f32),
            jax.ShapeDtypeStruct((T, BATCH, H), f32),
        ],
        scratch_shapes=[
            pltpu.VMEM((BATCH, H), f32),
            pltpu.VMEM((BATCH, H), f32),
            pltpu.VMEM((BATCH, H), f32),
            pltpu.VMEM((BATCH, H), f32),
        ],
    )(xwf, xwb, Whf, Whb)

    graph = _grn_sc(hf, hb)

    hf2 = hf.reshape(T, BATCH * H)
    hb2 = hb.reshape(T, BATCH * H)
    graph2 = graph.reshape(T, BATCH * OUT)
    WgL = Wg[:, :OUT].T
    WgG = Wg[:, OUT:].T
    WxT = Wx.T
    WgrT = Wgr.T
    WcT = Wc.T
    bg2 = bg.reshape(1, AIM)
    bf2 = bf.reshape(1, AIM)
    bc2 = bc.reshape(1, NC)

    logits = pl.pallas_call(
        _aim_kernel,
        grid=(BATCH,),
        in_specs=[
            pl.BlockSpec((T, H), lambda b: (0, b)),
            pl.BlockSpec((T, H), lambda b: (0, b)),
            pl.BlockSpec((T, OUT), lambda b: (0, b)),
            pl.BlockSpec((OUT, AIM), lambda b: (0, 0)),
            pl.BlockSpec((OUT, AIM), lambda b: (0, 0)),
            pl.BlockSpec((OUT, AIM), lambda b: (0, 0)),
            pl.BlockSpec((OUT, AIM), lambda b: (0, 0)),
            pl.BlockSpec((AIM, NC), lambda b: (0, 0)),
            pl.BlockSpec((1, AIM), lambda b: (0, 0)),
            pl.BlockSpec((1, AIM), lambda b: (0, 0)),
            pl.BlockSpec((1, NC), lambda b: (0, 0)),
        ],
        out_specs=pl.BlockSpec((1, T, NC), lambda b: (b, 0, 0)),
        out_shape=jax.ShapeDtypeStruct((BATCH, T, NC), f32),
    )(hf2, hb2, graph2, WgL, WgG, WxT, WgrT, WcT, bg2, bf2, bc2)

    return logits


# speedup vs baseline: 1.4931x; 1.4931x over previous
import functools

import jax
import jax.numpy as jnp
from jax.experimental import pallas as pl
from jax.experimental.pallas import tpu as pltpu
from jax.experimental.pallas import tpu_sc as plsc

TEXT_DIM = 512
AUDIO_DIM = 128
H = 128
OUT = 2 * H
T = 2048
BATCH = 8
WINDOW = 20
HOPS = 3
AIM = 256
NC = 7

BT = 256
NB = T // BT
BTA = 256
NBA = T // BTA
GBT = 256
NGB = T // GBT
PAD = 32


def _xw_kernel(xt_ref, xa_ref, Wt_ref, Wa_ref, b_ref, outf_ref, outb_ref):
    for b in range(BATCH):
        g = jnp.dot(xt_ref[b], Wt_ref[...], preferred_element_type=jnp.float32)
        g = g + jnp.dot(xa_ref[b], Wa_ref[...], preferred_element_type=jnp.float32)
        g = g + b_ref[...]
        outf_ref[:, b, :] = g[:, : 4 * H]
        outb_ref[:, b, :] = g[:, 4 * H:]


def _lstm_kernel(xwf_ref, xwb_ref, Whf_ref, Whb_ref, hf_ref, hb_ref,
                 hf_sc, hb_sc, cf_sc, cb_sc):
    i = pl.program_id(0)

    @pl.when(i == 0)
    def _():
        hf_sc[...] = jnp.zeros_like(hf_sc)
        hb_sc[...] = jnp.zeros_like(hb_sc)
        cf_sc[...] = jnp.zeros_like(cf_sc)
        cb_sc[...] = jnp.zeros_like(cb_sc)

    def sig(x):
        return 0.5 * jnp.tanh(0.5 * x) + 0.5

    def gates(g, c):
        ii = sig(g[:, 0:H])
        ff = sig(g[:, H:2 * H])
        uu = jnp.tanh(g[:, 2 * H:3 * H])
        oo = sig(g[:, 3 * H:4 * H])
        cn = ff * c + ii * uu
        hn = oo * jnp.tanh(cn)
        return hn, cn

    def body(k, carry):
        h_f, h_b, c_f, c_b = carry
        gf = jnp.dot(h_f.astype(jnp.bfloat16), Whf_ref[...],
                     preferred_element_type=jnp.float32) + xwf_ref[k]
        gb = jnp.dot(h_b.astype(jnp.bfloat16), Whb_ref[...],
                     preferred_element_type=jnp.float32) + xwb_ref[BT - 1 - k]
        hf_new, cf_new = gates(gf, c_f)
        hb_new, cb_new = gates(gb, c_b)
        hf_ref[k] = hf_new
        hb_ref[BT - 1 - k] = hb_new
        return (hf_new, hb_new, cf_new, cb_new)

    carry = (hf_sc[...], hb_sc[...], cf_sc[...], cb_sc[...])
    h_f, h_b, c_f, c_b = jax.lax.fori_loop(0, BT, body, carry, unroll=16)
    hf_sc[...] = h_f
    hb_sc[...] = h_b
    cf_sc[...] = c_f
    cb_sc[...] = c_b


GB = 256
HALO = 64
ROWS = GB + HALO
FH = 128
NTB = T // GB
UNITS_PER_WORKER = (BATCH * NTB * 2) // 32


def _grn_sc_kernel(hf_hbm, hb_hbm, out_hbm, bufA, bufB, acc, invd):
    cid = jax.lax.axis_index("c")
    sid = jax.lax.axis_index("s")
    wid = sid * 2 + cid

    for u in range(UNITS_PER_WORKER):
        unit = wid * UNITS_PER_WORKER + u
        half = unit % 2
        tb = (unit // 2) % NTB
        b = unit // (2 * NTB)
        r0 = tb * GB

        @pl.when(jnp.logical_and(half == 0, tb == 0))
        def _():
            pltpu.sync_copy(hf_hbm.at[pl.ds(0, GB), b, :],
                            bufA.at[pl.ds(HALO, GB), :])

        @pl.when(jnp.logical_and(half == 1, tb == 0))
        def _():
            pltpu.sync_copy(hb_hbm.at[pl.ds(0, GB), b, :],
                            bufA.at[pl.ds(HALO, GB), :])

        @pl.when(jnp.logical_and(half == 0, tb != 0))
        def _():
            pltpu.sync_copy(hf_hbm.at[pl.ds(r0 - HALO, ROWS), b, :],
                            bufA.at[:])

        @pl.when(jnp.logical_and(half == 1, tb != 0))
        def _():
            pltpu.sync_copy(hb_hbm.at[pl.ds(r0 - HALO, ROWS), b, :],
                            bufA.at[:])

        @pl.when(tb == 0)
        def _():
            def zrow(j, _):
                for c in range(FH // 16):
                    bufA[j, pl.ds(c * 16, 16)] = jnp.zeros((16,), jnp.float32)
                return 0
            jax.lax.fori_loop(0, HALO, zrow, 0)

        def fill(j, _):
            invd[j] = jnp.float32(1.0 / (WINDOW + 1))
            return 0
        jax.lax.fori_loop(0, ROWS, fill, 0)

        @pl.when(tb == 0)
        def _():
            for g in range(WINDOW):
                invd[HALO + g] = jnp.float32(1.0 / (g + 1))

        def initacc(j, _):
            for c in range(FH // 16):
                acc[j, pl.ds(c * 16, 16)] = bufA[HALO + j, pl.ds(c * 16, 16)]
            return 0
        jax.lax.fori_loop(0, GB, initacc, 0)

        bufs = [bufA, bufB, bufA, bufB]
        for hop in range(HOPS):
            X, Y = bufs[hop], bufs[hop + 1]

            def presum(j, S):
                return tuple(S[c] + X[j, pl.ds(c * 16, 16)]
                             for c in range(FH // 16))

            S0 = tuple(jnp.zeros((16,), jnp.float32) for _ in range(FH // 16))
            S = jax.lax.fori_loop(0, WINDOW, presum, S0)

            def row(j, S):
                inv = invd[j]
                out = []
                for c in range(FH // 16):
                    s = S[c] + X[j, pl.ds(c * 16, 16)]
                    Y[j, pl.ds(c * 16, 16)] = s * inv
                    out.append(s - X[j - WINDOW, pl.ds(c * 16, 16)])
                return tuple(out)
            jax.lax.fori_loop(WINDOW, ROWS, row, S)

            def zy(j, _):
                for c in range(FH // 16):
                    Y[j, pl.ds(c * 16, 16)] = jnp.zeros((16,), jnp.float32)
                return 0
            jax.lax.fori_loop(0, WINDOW, zy, 0)

            def addacc(j, _):
                for c in range(FH // 16):
                    acc[j, pl.ds(c * 16, 16)] += Y[HALO + j, pl.ds(c * 16, 16)]
                return 0
            jax.lax.fori_loop(0, GB, addacc, 0)

        pltpu.sync_copy(acc.at[:],
                        out_hbm.at[pl.ds(r0, GB), b, pl.ds(half * FH, FH)])


_grn_sc = functools.partial(
    pl.kernel,
    mesh=plsc.VectorSubcoreMesh(core_axis_name="c", subcore_axis_name="s"),
    out_type=jax.ShapeDtypeStruct((T, BATCH, OUT), jnp.float32),
    scratch_types=[
        pltpu.VMEM((ROWS, FH), jnp.float32),
        pltpu.VMEM((ROWS, FH), jnp.float32),
        pltpu.VMEM((GB, FH), jnp.float32),
        pltpu.SMEM((ROWS,), jnp.float32),
    ],
)(_grn_sc_kernel)


def _aim_kernel(hf_ref, hb_ref, graph_ref, WgL_ref, WgG_ref, Wx_ref, Wgr_ref,
                Wc_ref, bg_ref, bfu_ref, bc_ref, out_ref):
    lstm = jnp.concatenate([hf_ref[...], hb_ref[...]], axis=1)
    graph = graph_ref[...] * (1.0 / (HOPS + 1))

    gate = jax.nn.sigmoid(
        jnp.dot(lstm, WgL_ref[...], preferred_element_type=jnp.float32)
        + jnp.dot(graph, WgG_ref[...], preferred_element_type=jnp.float32)
        + bg_ref[...])
    fused = jnp.tanh(
        gate * jnp.dot(lstm, Wx_ref[...], preferred_element_type=jnp.float32)
        + (1.0 - gate) * jnp.dot(graph, Wgr_ref[...], preferred_element_type=jnp.float32)
        + bfu_ref[...])
    out_ref[0] = jnp.dot(fused, Wc_ref[...], preferred_element_type=jnp.float32) + bc_ref[...]


def kernel(text_embeds, audio_feats, speaker_ids, W_ih_f, W_hh_f, b_f,
           W_ih_b, W_hh_b, b_b, Wg, bg, Wx, Wgr, bf, Wc, bc):
    f32 = jnp.float32
    Wt = jnp.concatenate([W_ih_f[:, :TEXT_DIM], W_ih_b[:, :TEXT_DIM]], axis=0).T
    Wa = jnp.concatenate([W_ih_f[:, TEXT_DIM:], W_ih_b[:, TEXT_DIM:]], axis=0).T
    bcat = jnp.concatenate([b_f, b_b]).reshape(1, 8 * H)
    Whf = W_hh_f.T.astype(jnp.bfloat16)
    Whb = W_hh_b.T.astype(jnp.bfloat16)

    xwf, xwb = pl.pallas_call(
        _xw_kernel,
        grid=(NBA,),
        in_specs=[
            pl.BlockSpec((BATCH, BTA, TEXT_DIM), lambda i: (0, i, 0)),
            pl.BlockSpec((BATCH, BTA, AUDIO_DIM), lambda i: (0, i, 0)),
            pl.BlockSpec((TEXT_DIM, 8 * H), lambda i: (0, 0)),
            pl.BlockSpec((AUDIO_DIM, 8 * H), lambda i: (0, 0)),
            pl.BlockSpec((1, 8 * H), lambda i: (0, 0)),
        ],
        out_specs=[
            pl.BlockSpec((BTA, BATCH, 4 * H), lambda i: (i, 0, 0)),
            pl.BlockSpec((BTA, BATCH, 4 * H), lambda i: (i, 0, 0)),
        ],
        out_shape=[
            jax.ShapeDtypeStruct((T, BATCH, 4 * H), f32),
            jax.ShapeDtypeStruct((T, BATCH, 4 * H), f32),
        ],
    )(text_embeds, audio_feats, Wt, Wa, bcat)

    hf, hb = pl.pallas_call(
        _lstm_kernel,
        grid=(NB,),
        in_specs=[
            pl.BlockSpec((BT, BATCH, 4 * H), lambda i: (i, 0, 0)),
            pl.BlockSpec((BT, BATCH, 4 * H), lambda i: (NB - 1 - i, 0, 0)),
            pl.BlockSpec((H, 4 * H), lambda i: (0, 0)),
            pl.BlockSpec((H, 4 * H), lambda i: (0, 0)),
        ],
        out_specs=[
            pl.BlockSpec((BT, BATCH, H), lambda i: (i, 0, 0)),
            pl.BlockSpec((BT, BATCH, H), lambda i: (NB - 1 - i, 0, 0)),
        ],
        out_shape=[
            jax.ShapeDtypeStruct((T, BATCH, H), f32),
            jax.ShapeDtypeStruct((T, BATCH, H), f32),
        ],
        scratch_shapes=[
            pltpu.VMEM((BATCH, H), f32),
            pltpu.VMEM((BATCH, H), f32),
            pltpu.VMEM((BATCH, H), f32),
            pltpu.VMEM((BATCH, H), f32),
        ],
    )(xwf, xwb, Whf, Whb)

    graph = _grn_sc(hf, hb)

    hf2 = hf.reshape(T, BATCH * H)
    hb2 = hb.reshape(T, BATCH * H)
    graph2 = graph.reshape(T, BATCH * OUT)
    WgL = Wg[:, :OUT].T
    WgG = Wg[:, OUT:].T
    WxT = Wx.T
    WgrT = Wgr.T
    WcT = Wc.T
    bg2 = bg.reshape(1, AIM)
    bf2 = bf.reshape(1, AIM)
    bc2 = bc.reshape(1, NC)

    logits = pl.pallas_call(
        _aim_kernel,
        grid=(BATCH,),
        in_specs=[
            pl.BlockSpec((T, H), lambda b: (0, b)),
            pl.BlockSpec((T, H), lambda b: (0, b)),
            pl.BlockSpec((T, OUT), lambda b: (0, b)),
            pl.BlockSpec((OUT, AIM), lambda b: (0, 0)),
            pl.BlockSpec((OUT, AIM), lambda b: (0, 0)),
            pl.BlockSpec((OUT, AIM), lambda b: (0, 0)),
            pl.BlockSpec((OUT, AIM), lambda b: (0, 0)),
            pl.BlockSpec((AIM, NC), lambda b: (0, 0)),
            pl.BlockSpec((1, AIM), lambda b: (0, 0)),
            pl.BlockSpec((1, AIM), lambda b: (0, 0)),
            pl.BlockSpec((1, NC), lambda b: (0, 0)),
        ],
        out_specs=pl.BlockSpec((1, T, NC), lambda b: (b, 0, 0)),
        out_shape=jax.ShapeDtypeStruct((BATCH, T, NC), f32),
    )(hf2, hb2, graph2, WgL, WgG, WxT, WgrT, WcT, bg2, bf2, bc2)

    return logits
